# Initial kernel scaffold; baseline (speedup 1.0000x reference)
import jax
import jax.numpy as jnp
from jax.experimental import pallas as pl


def kernel(left_buf, left_count, right_buf, right_count, subs):
    raise NotImplementedError("write your pallas kernel here")


# speedup vs baseline: 1.1252x; 1.1252x over previous
import jax
import jax.numpy as jnp
from jax.experimental import pallas as pl
from jax.experimental.pallas import tpu as pltpu

B, N, MO, D = 8, 512, 64, 64
NB = 16


def _round_half_even_nonneg(x):
    i = x.astype(jnp.int32)
    frac = x - i.astype(jnp.float32)
    odd = (i & 1) == 1
    up = (frac > 0.5) | ((frac == 0.5) & odd)
    return i + jnp.where(up, 1, 0)


def _body(lc_ref, rc_ref, subs_ref, lb_ref, rb_ref, out_ref, cnt_ref):
    m2 = jax.lax.broadcasted_iota(jnp.int32, (MO, MO), 0)
    k2 = jax.lax.broadcasted_iota(jnp.int32, (MO, MO), 1)

    for s in range(NB):
        lc = lc_ref[0, 0, 0, s]
        rc = rc_ref[0, 0, 0, s]
        is_after = subs_ref[0, 0, 0, s] == 1
        fcf = jnp.where(is_after, rc, lc)
        scf = jnp.where(is_after, lc, rc)
        fc = _round_half_even_nonneg(fcf)
        sc = _round_half_even_nonneg(scf)

        p_first = ((m2 == k2) & (m2 < fc)).astype(jnp.float32)
        p_second = ((m2 - k2 == fc) & (k2 < sc)).astype(jnp.float32)
        p_left = jnp.where(is_after, p_second, p_first)
        p_right = jnp.where(is_after, p_first, p_second)

        out_ref[0, s] = (
            jax.lax.dot(p_left, lb_ref[0, s],
                        precision=jax.lax.Precision.HIGHEST,
                        preferred_element_type=jnp.float32)
            + jax.lax.dot(p_right, rb_ref[0, s],
                          precision=jax.lax.Precision.HIGHEST,
                          preferred_element_type=jnp.float32)
        )
        cnt_ref[0, 0, 0, s] = jnp.minimum(lc + rc, float(MO))


def kernel(left_buf, left_count, right_buf, right_count, subs):
    nj = N // NB
    lc4 = left_count.reshape(B, nj, 1, NB)
    rc4 = right_count.reshape(B, nj, 1, NB)
    subs4 = subs.reshape(B, nj, 1, NB)

    buf_spec = pl.BlockSpec((1, NB, MO, D), lambda b, j: (b, j, 0, 0))
    smem_spec = pl.BlockSpec((1, 1, 1, NB), lambda b, j: (b, j, 0, 0),
                             memory_space=pltpu.SMEM)

    out_buf, out_cnt = pl.pallas_call(
        _body,
        grid=(B, nj),
        in_specs=[smem_spec, smem_spec, smem_spec, buf_spec, buf_spec],
        out_specs=[buf_spec, smem_spec],
        out_shape=[
            jax.ShapeDtypeStruct((B, N, MO, D), jnp.float32),
            jax.ShapeDtypeStruct((B, nj, 1, NB), jnp.float32),
        ],
    )(lc4, rc4, subs4, left_buf, right_buf)

    return out_buf, out_cnt.reshape(B, N)
